# Initial kernel scaffold; baseline (speedup 1.0000x reference)
import jax
import jax.numpy as jnp
from jax.experimental import pallas as pl


def kernel(xs, xg, edge_src_g, edge_dst_s, edge_weight, W_s_pre, W_g_pre, W_gs, W_sg, W_s_post, W_g_post):
    raise NotImplementedError("write your pallas kernel here")


# speedup vs baseline: 3.2226x; 3.2226x over previous
import functools

import jax
import jax.numpy as jnp
from jax import lax
from jax.experimental import pallas as pl
from jax.experimental.pallas import tpu as pltpu
from jax.experimental.pallas import tpu_sc as plsc

NS, NG, D = 50000, 12500, 128
CH = 32
NCHUNK = D // CH
NS_PAD = 51200
NG_PAD = 12800
N_TILES = 16
EC = 512
EPT = 62 * EC
E_PAD = EPT * N_TILES
N_EC = EPT // EC
ZROWS = NS_PAD // N_TILES
ZROWS_G = NG_PAD // N_TILES

_MESH = plsc.VectorSubcoreMesh(
    core_axis_name="c", subcore_axis_name="s", num_cores=2, num_subcores=16
)


def _pre_body(x_ref, w_ref, out_ref):
    p = jnp.dot(x_ref[...], w_ref[...], preferred_element_type=jnp.float32)
    for c in range(NCHUNK):
        out_ref[c] = p[:, c * CH:(c + 1) * CH]


def _pre_matmul(x, w):
    n = x.shape[0]
    b = 512
    grid = (pl.cdiv(n, b),)
    return pl.pallas_call(
        _pre_body,
        grid=grid,
        in_specs=[
            pl.BlockSpec((b, D), lambda i: (i, 0)),
            pl.BlockSpec((D, D), lambda i: (0, 0)),
        ],
        out_specs=pl.BlockSpec((NCHUNK, b, CH), lambda i: (0, i, 0)),
        out_shape=jax.ShapeDtypeStruct((NCHUNK, n, CH), jnp.float32),
    )(x, w)


def _fold_body(wgs_ref, wsg_ref, wsp_ref, wgp_ref, wbs_ref, wbg_ref):
    wbs_ref[...] = jnp.dot(wgs_ref[...], wsp_ref[D:, :],
                           preferred_element_type=jnp.float32)
    wbg_ref[...] = jnp.dot(wsg_ref[...], wgp_ref[D:, :],
                           preferred_element_type=jnp.float32)


def _fold_weights(w_gs, w_sg, w_s_post, w_g_post):
    return pl.pallas_call(
        _fold_body,
        out_shape=(
            jax.ShapeDtypeStruct((D, D), jnp.float32),
            jax.ShapeDtypeStruct((D, D), jnp.float32),
        ),
    )(w_gs, w_sg, w_s_post, w_g_post)


def _post_body(xp_ref, agg_ref, p1_ref, wb_ref, out_ref):
    acc = jnp.dot(xp_ref[0], p1_ref[0:CH, :], preferred_element_type=jnp.float32)
    for c in range(NCHUNK):
        if c:
            acc += jnp.dot(xp_ref[c], p1_ref[c * CH:(c + 1) * CH, :],
                           preferred_element_type=jnp.float32)
        acc += jnp.dot(agg_ref[c], wb_ref[c * CH:(c + 1) * CH, :],
                       preferred_element_type=jnp.float32)
    out_ref[...] = acc


def _post_matmul(xp, agg, p1, wb, n):
    b = 512
    grid = (pl.cdiv(n, b),)
    return pl.pallas_call(
        _post_body,
        grid=grid,
        in_specs=[
            pl.BlockSpec((NCHUNK, b, CH), lambda i: (0, i, 0)),
            pl.BlockSpec((NCHUNK, b, CH), lambda i: (0, i, 0)),
            pl.BlockSpec((D, D), lambda i: (0, 0)),
            pl.BlockSpec((D, D), lambda i: (0, 0)),
        ],
        out_specs=pl.BlockSpec((b, D), lambda i: (i, 0)),
        out_shape=jax.ShapeDtypeStruct((n, D), jnp.float32),
    )(xp, agg, p1, wb)


def _sc_body(xs_t, xg_t, src2d, dst2d, w2d, zeros_h,
             out_s, out_g,
             gidx_v, sidx_v, w_v, rows_v, acc, sem):
    core = lax.axis_index("c")
    sid = lax.axis_index("s")

    for direction in range(2):
        if direction == 0:
            table, trows, gat2d, sct2d = xg_t, NG, src2d, dst2d
            out, npad, nz = out_s, NS_PAD, ZROWS
        else:
            table, trows, gat2d, sct2d = xs_t, NS, dst2d, src2d
            out, npad, nz = out_g, NG_PAD, ZROWS_G
        for k in range(2):
            chunk = core * 2 + k
            off = chunk * trows

            plsc.subcore_barrier()
            pltpu.sync_copy(zeros_h.at[pl.ds(0, nz)],
                            acc.at[pl.ds(sid * nz, nz)])
            plsc.subcore_barrier()

            @pl.loop(0, N_EC)
            def _sweep(ci, _direction=direction, _table=table, _off=off,
                       _gat2d=gat2d, _sct2d=sct2d):
                base2 = sid * (EPT // 128) + ci * (EC // 128)
                pltpu.sync_copy(_gat2d.at[pl.ds(base2, EC // 128)], gidx_v)
                pltpu.sync_copy(_sct2d.at[pl.ds(base2, EC // 128)], sidx_v)
                pltpu.sync_copy(w2d.at[pl.ds(base2, EC // 128)], w_v)
                for j in range(EC // 128):
                    for v in range(8):
                        gidx_v[j, pl.ds(v * 16, 16)] = (
                            gidx_v[j, pl.ds(v * 16, 16)] + _off)
                cps = [
                    pltpu.async_copy(
                        _table.at[gidx_v.at[j]],
                        rows_v.at[pl.ds(j * 128, 128)], sem)
                    for j in range(EC // 128)
                ]
                for cp in cps:
                    cp.wait()
                for j in range(EC // 128):
                    @pl.loop(0, 8)
                    def _scale(g, _j=j):
                        wv = w_v[_j, pl.ds(g * 16, 16)]
                        base = _j * 128 + g * 16
                        for e in range(16):
                            r = base + e
                            ws = wv[e]
                            rows_v[r, pl.ds(0, 16)] = (
                                rows_v[r, pl.ds(0, 16)] * ws)
                            rows_v[r, pl.ds(16, 16)] = (
                                rows_v[r, pl.ds(16, 16)] * ws)
                for j in range(EC // 128):
                    pltpu.sync_copy(rows_v.at[pl.ds(j * 128, 128)],
                                    acc.at[sidx_v.at[j]], add=True)

            plsc.subcore_barrier()
            pltpu.sync_copy(
                acc.at[pl.ds(sid * nz, nz)],
                out.at[pl.ds(chunk * npad + sid * nz, nz)])


def _sc_spmm(xs_t_flat, xg_t_flat, src2d, dst2d, w2d, zeros_h):
    f = pl.kernel(
        _sc_body,
        out_type=(
            jax.ShapeDtypeStruct((NCHUNK * NS_PAD, CH), jnp.float32),
            jax.ShapeDtypeStruct((NCHUNK * NG_PAD, CH), jnp.float32),
        ),
        mesh=_MESH,
        compiler_params=pltpu.CompilerParams(use_tc_tiling_on_sc=False),
        scratch_types=[
            pltpu.VMEM((EC // 128, 128), jnp.int32),
            pltpu.VMEM((EC // 128, 128), jnp.int32),
            pltpu.VMEM((EC // 128, 128), jnp.float32),
            pltpu.VMEM((EC, CH), jnp.float32),
            pltpu.VMEM_SHARED((NS_PAD, CH), jnp.float32),
            pltpu.SemaphoreType.DMA,
        ],
    )
    return f(xs_t_flat, xg_t_flat, src2d, dst2d, w2d, zeros_h)


@jax.jit
def kernel(xs, xg, edge_src_g, edge_dst_s, edge_weight,
           W_s_pre, W_g_pre, W_gs, W_sg, W_s_post, W_g_post):
    e = edge_src_g.shape[0]
    pad = E_PAD - e
    pad_i = jnp.arange(pad, dtype=jnp.int32)
    src2d = jnp.concatenate([edge_src_g, pad_i % NG]).reshape(-1, 128)
    dst2d = jnp.concatenate([edge_dst_s, pad_i % NS]).reshape(-1, 128)
    w2d = jnp.concatenate(
        [edge_weight, jnp.zeros((pad,), jnp.float32)]).reshape(-1, 128)
    zeros_h = jnp.zeros((ZROWS, CH), jnp.float32)

    xs_t = _pre_matmul(xs, W_s_pre)
    xg_t = _pre_matmul(xg, W_g_pre)
    wb_s, wb_g = _fold_weights(W_gs, W_sg, W_s_post, W_g_post)

    agg_s_f, agg_g_f = _sc_spmm(
        xs_t.reshape(-1, CH), xg_t.reshape(-1, CH), src2d, dst2d, w2d, zeros_h)
    agg_s = agg_s_f.reshape(NCHUNK, NS_PAD, CH)
    agg_g = agg_g_f.reshape(NCHUNK, NG_PAD, CH)

    xs_new = _post_matmul(xs_t, agg_s, W_s_post[:D], wb_s, NS)
    xg_new = _post_matmul(xg_t, agg_g, W_g_post[:D], wb_g, NG)
    return xs_new, xg_new


# speedup vs baseline: 5.6622x; 1.7570x over previous
import functools

import jax
import jax.numpy as jnp
from jax import lax
from jax.experimental import pallas as pl
from jax.experimental.pallas import tpu as pltpu
from jax.experimental.pallas import tpu_sc as plsc

NS, NG, D = 50000, 12500, 128
CH = 32
NCHUNK = D // CH
NS_PAD = 50048
NG_PAD = 12800
N_TILES = 16
EC = 256
N_EC = 126
EPT = N_EC * EC
E_PAD = EPT * N_TILES
IRPT = EPT // 128
ZROWS = NS_PAD // N_TILES
ZROWS_G = NG_PAD // N_TILES
GBYTES = 128 * CH * 4

_MESH = plsc.VectorSubcoreMesh(
    core_axis_name="c", subcore_axis_name="s", num_cores=2, num_subcores=16
)


def _pre_body(x_ref, w_ref, out_ref):
    p = jnp.dot(x_ref[...], w_ref[...], preferred_element_type=jnp.float32)
    for c in range(NCHUNK):
        out_ref[c] = p[:, c * CH:(c + 1) * CH]


def _pre_matmul(x, w):
    n = x.shape[0]
    b = 512
    grid = (pl.cdiv(n, b),)
    return pl.pallas_call(
        _pre_body,
        grid=grid,
        in_specs=[
            pl.BlockSpec((b, D), lambda i: (i, 0)),
            pl.BlockSpec((D, D), lambda i: (0, 0)),
        ],
        out_specs=pl.BlockSpec((NCHUNK, b, CH), lambda i: (0, i, 0)),
        out_shape=jax.ShapeDtypeStruct((NCHUNK, n, CH), jnp.float32),
    )(x, w)


def _fold_body(wgs_ref, wsg_ref, wsp_ref, wgp_ref, wbs_ref, wbg_ref):
    wbs_ref[...] = jnp.dot(wgs_ref[...], wsp_ref[D:, :],
                           preferred_element_type=jnp.float32)
    wbg_ref[...] = jnp.dot(wsg_ref[...], wgp_ref[D:, :],
                           preferred_element_type=jnp.float32)


def _fold_weights(w_gs, w_sg, w_s_post, w_g_post):
    return pl.pallas_call(
        _fold_body,
        out_shape=(
            jax.ShapeDtypeStruct((D, D), jnp.float32),
            jax.ShapeDtypeStruct((D, D), jnp.float32),
        ),
    )(w_gs, w_sg, w_s_post, w_g_post)


def _post_body(xp_ref, agg_ref, p1_ref, wb_ref, out_ref):
    acc = jnp.dot(xp_ref[0], p1_ref[0:CH, :], preferred_element_type=jnp.float32)
    for c in range(NCHUNK):
        if c:
            acc += jnp.dot(xp_ref[c], p1_ref[c * CH:(c + 1) * CH, :],
                           preferred_element_type=jnp.float32)
        acc += jnp.dot(agg_ref[c], wb_ref[c * CH:(c + 1) * CH, :],
                       preferred_element_type=jnp.float32)
    out_ref[...] = acc


def _post_matmul(xp, agg, p1, wb, n):
    b = 512
    grid = (pl.cdiv(n, b),)
    return pl.pallas_call(
        _post_body,
        grid=grid,
        in_specs=[
            pl.BlockSpec((NCHUNK, b, CH), lambda i: (0, i, 0)),
            pl.BlockSpec((NCHUNK, b, CH), lambda i: (0, i, 0)),
            pl.BlockSpec((D, D), lambda i: (0, 0)),
            pl.BlockSpec((D, D), lambda i: (0, 0)),
        ],
        out_specs=pl.BlockSpec((b, D), lambda i: (i, 0)),
        out_shape=jax.ShapeDtypeStruct((n, D), jnp.float32),
    )(xp, agg, p1, wb)


def _sc_body(xs_t, xg_t, src2d, dst2d, w2d, zeros_h,
             out_s, out_g,
             rows, gidx, sidx, wvb, acc,
             si0, si1, sg0, sg1, sg2, sw0, sw1, sw2):
    core = lax.axis_index("c")
    sid = lax.axis_index("s")
    si = [si0, si1]
    sg = [sg0, sg1, sg2]
    sw = [sw0, sw1, sw2]

    for direction in range(2):
        if direction == 0:
            table, trows, gat2d, sct2d = xg_t, NG, src2d, dst2d
            out, npad, nz = out_s, NS_PAD, ZROWS
        else:
            table, trows, gat2d, sct2d = xs_t, NS, dst2d, src2d
            out, npad, nz = out_g, NG_PAD, ZROWS_G

        def fire_i(c, u, _g2=gat2d, _s2=sct2d):
            b2 = sid * IRPT + c * (EC // 128)
            s = si[u % 2]
            pltpu.async_copy(_g2.at[pl.ds(b2, 2)], gidx.at[u], s)
            pltpu.async_copy(_s2.at[pl.ds(b2, 2)], sidx.at[u], s)
            pltpu.async_copy(w2d.at[pl.ds(b2, 2)], wvb.at[u], s)

        def wait_i(u, _g2=gat2d, _s2=sct2d):
            s = si[u % 2]
            pltpu.make_async_copy(_g2.at[pl.ds(0, 2)], gidx.at[u], s).wait()
            pltpu.make_async_copy(_s2.at[pl.ds(0, 2)], sidx.at[u], s).wait()
            pltpu.make_async_copy(w2d.at[pl.ds(0, 2)], wvb.at[u], s).wait()

        def add_off(u, _off):
            for j in range(EC // 128):
                for v in range(8):
                    gidx[u, j, pl.ds(v * 16, 16)] = (
                        gidx[u, j, pl.ds(v * 16, 16)] + _off)

        def fire_g(u, _table=table):
            b3 = u % 3
            for j in range(EC // 128):
                pltpu.async_copy(_table.at[gidx.at[u, j]],
                                 rows.at[b3, pl.ds(j * 128, 128)], sg[b3])

        def wait_g(u, _table=table):
            b3 = u % 3
            for j in range(EC // 128):
                pltpu.make_async_copy(
                    _table.at[gidx.at[u, j]],
                    rows.at[b3, pl.ds(j * 128, 128)], sg[b3]).wait()

        def scale(u):
            b3 = u % 3

            @pl.loop(0, EC // 16)
            def _scale(g16, _u=u, _b3=b3):
                jrow = g16 >> 3
                lo = (g16 & 7) * 16
                w16 = wvb[_u, jrow, pl.ds(lo, 16)]
                base = g16 * 16
                for e in range(16):
                    ws = w16[e]
                    rows[_b3, base + e, pl.ds(0, 16)] = (
                        rows[_b3, base + e, pl.ds(0, 16)] * ws)
                    rows[_b3, base + e, pl.ds(16, 16)] = (
                        rows[_b3, base + e, pl.ds(16, 16)] * ws)

        def fire_w(u):
            b3 = u % 3
            for j in range(EC // 128):
                pltpu.async_copy(rows.at[b3, pl.ds(j * 128, 128)],
                                 acc.at[sidx.at[u, j]], sw[b3], add=True)

        def wait_w(u):
            b3 = u % 3
            for j in range(EC // 128):
                pltpu.make_async_copy(rows.at[b3, pl.ds(j * 128, 128)],
                                      acc.at[sidx.at[u, j]], sw[b3]).wait()

        for k in range(2):
            chunk = core * 2 + k
            off = chunk * trows

            plsc.subcore_barrier()
            pltpu.sync_copy(zeros_h.at[pl.ds(0, nz)],
                            acc.at[pl.ds(sid * nz, nz)])
            plsc.subcore_barrier()

            fire_i(0, 0)
            fire_i(1, 1)
            pltpu.async_copy(zeros_h.at[pl.ds(0, EC)], rows.at[1], sw[1])
            pltpu.async_copy(zeros_h.at[pl.ds(0, EC)], rows.at[2], sw[2])
            wait_i(0)
            add_off(0, off)
            fire_g(0)

            @pl.loop(0, N_EC // 6)
            def _grp(g, _off=off):
                for u in range(6):
                    c = g * 6 + u
                    un1, un2 = (u + 1) % 6, (u + 2) % 6
                    wait_i(un1)
                    add_off(un1, _off)
                    wait_w((u + 4) % 6)
                    fire_g(un1)
                    fire_i(jnp.minimum(c + 2, N_EC - 1), un2)
                    wait_g(u)
                    scale(u)
                    fire_w(u)

            wait_w(4)
            wait_w(5)
            wait_g(0)
            wait_i(1)

            plsc.subcore_barrier()
            pltpu.sync_copy(
                acc.at[pl.ds(sid * nz, nz)],
                out.at[pl.ds(chunk * npad + sid * nz, nz)])


def _sc_spmm(xs_t_flat, xg_t_flat, src2d, dst2d, w2d, zeros_h):
    f = pl.kernel(
        _sc_body,
        out_type=(
            jax.ShapeDtypeStruct((NCHUNK * NS_PAD, CH), jnp.float32),
            jax.ShapeDtypeStruct((NCHUNK * NG_PAD, CH), jnp.float32),
        ),
        mesh=_MESH,
        compiler_params=pltpu.CompilerParams(use_tc_tiling_on_sc=False),
        scratch_types=[
            pltpu.VMEM((3, EC, CH), jnp.float32),
            pltpu.VMEM((6, EC // 128, 128), jnp.int32),
            pltpu.VMEM((6, EC // 128, 128), jnp.int32),
            pltpu.VMEM((6, EC // 128, 128), jnp.float32),
            pltpu.VMEM_SHARED((NS_PAD, CH), jnp.float32),
            pltpu.SemaphoreType.DMA,
            pltpu.SemaphoreType.DMA,
            pltpu.SemaphoreType.DMA,
            pltpu.SemaphoreType.DMA,
            pltpu.SemaphoreType.DMA,
            pltpu.SemaphoreType.DMA,
            pltpu.SemaphoreType.DMA,
            pltpu.SemaphoreType.DMA,
        ],
    )
    return f(xs_t_flat, xg_t_flat, src2d, dst2d, w2d, zeros_h)


@jax.jit
def kernel(xs, xg, edge_src_g, edge_dst_s, edge_weight,
           W_s_pre, W_g_pre, W_gs, W_sg, W_s_post, W_g_post):
    e = edge_src_g.shape[0]
    pad = E_PAD - e
    pad_i = jnp.arange(pad, dtype=jnp.int32)
    src2d = jnp.concatenate([edge_src_g, pad_i % NG]).reshape(-1, 128)
    dst2d = jnp.concatenate([edge_dst_s, pad_i % NS]).reshape(-1, 128)
    w2d = jnp.concatenate(
        [edge_weight, jnp.zeros((pad,), jnp.float32)]).reshape(-1, 128)
    zeros_h = jnp.zeros((ZROWS, CH), jnp.float32)

    xs_t = _pre_matmul(xs, W_s_pre)
    xg_t = _pre_matmul(xg, W_g_pre)
    wb_s, wb_g = _fold_weights(W_gs, W_sg, W_s_post, W_g_post)

    agg_s_f, agg_g_f = _sc_spmm(
        xs_t.reshape(-1, CH), xg_t.reshape(-1, CH), src2d, dst2d, w2d, zeros_h)
    agg_s = agg_s_f.reshape(NCHUNK, NS_PAD, CH)
    agg_g = agg_g_f.reshape(NCHUNK, NG_PAD, CH)

    xs_new = _post_matmul(xs_t, agg_s, W_s_post[:D], wb_s, NS)
    xg_new = _post_matmul(xg_t, agg_g, W_g_post[:D], wb_g, NG)
    return xs_new, xg_new


# speedup vs baseline: 6.8699x; 1.2133x over previous
import functools

import jax
import jax.numpy as jnp
from jax import lax
from jax.experimental import pallas as pl
from jax.experimental.pallas import tpu as pltpu
from jax.experimental.pallas import tpu_sc as plsc

NS, NG, D = 50000, 12500, 128
CH = 32
NCHUNK = D // CH
NS_PAD = 50048
NG_PAD = 12800
N_TILES = 16
EC = 256
N_EC = 126
EPT = N_EC * EC
E_PAD = EPT * N_TILES
IRPT = EPT // 128
ZROWS = NS_PAD // N_TILES
ZROWS_G = NG_PAD // N_TILES
GBYTES = 128 * CH * 4

_MESH = plsc.VectorSubcoreMesh(
    core_axis_name="c", subcore_axis_name="s", num_cores=2, num_subcores=16
)


def _pre_body(x_ref, w_ref, out_ref):
    p = jnp.dot(x_ref[...], w_ref[...], preferred_element_type=jnp.float32)
    for c in range(NCHUNK):
        out_ref[c] = p[:, c * CH:(c + 1) * CH]


def _pre_matmul(x, w):
    n = x.shape[0]
    b = 512
    grid = (pl.cdiv(n, b),)
    return pl.pallas_call(
        _pre_body,
        grid=grid,
        in_specs=[
            pl.BlockSpec((b, D), lambda i: (i, 0)),
            pl.BlockSpec((D, D), lambda i: (0, 0)),
        ],
        out_specs=pl.BlockSpec((NCHUNK, b, CH), lambda i: (0, i, 0)),
        out_shape=jax.ShapeDtypeStruct((NCHUNK, n, CH), jnp.float32),
    )(x, w)


def _fold_body(wgs_ref, wsg_ref, wsp_ref, wgp_ref, wbs_ref, wbg_ref):
    wbs_ref[...] = jnp.dot(wgs_ref[...], wsp_ref[D:, :],
                           preferred_element_type=jnp.float32)
    wbg_ref[...] = jnp.dot(wsg_ref[...], wgp_ref[D:, :],
                           preferred_element_type=jnp.float32)


def _fold_weights(w_gs, w_sg, w_s_post, w_g_post):
    return pl.pallas_call(
        _fold_body,
        out_shape=(
            jax.ShapeDtypeStruct((D, D), jnp.float32),
            jax.ShapeDtypeStruct((D, D), jnp.float32),
        ),
    )(w_gs, w_sg, w_s_post, w_g_post)


def _post_body(xp_ref, agg_ref, p1_ref, wb_ref, out_ref):
    acc = jnp.dot(xp_ref[0], p1_ref[0:CH, :], preferred_element_type=jnp.float32)
    for c in range(NCHUNK):
        if c:
            acc += jnp.dot(xp_ref[c], p1_ref[c * CH:(c + 1) * CH, :],
                           preferred_element_type=jnp.float32)
        acc += jnp.dot(agg_ref[c], wb_ref[c * CH:(c + 1) * CH, :],
                       preferred_element_type=jnp.float32)
    out_ref[...] = acc


def _post_matmul(xp, agg, p1, wb, n):
    b = 512
    grid = (pl.cdiv(n, b),)
    return pl.pallas_call(
        _post_body,
        grid=grid,
        in_specs=[
            pl.BlockSpec((NCHUNK, b, CH), lambda i: (0, i, 0)),
            pl.BlockSpec((NCHUNK, b, CH), lambda i: (0, i, 0)),
            pl.BlockSpec((D, D), lambda i: (0, 0)),
            pl.BlockSpec((D, D), lambda i: (0, 0)),
        ],
        out_specs=pl.BlockSpec((b, D), lambda i: (i, 0)),
        out_shape=jax.ShapeDtypeStruct((n, D), jnp.float32),
    )(xp, agg, p1, wb)


def _sc_body_dir(trows, npad, nz,
                 table, gat2d, sct2d, w2d, zeros_h,
                 out,
                 rows, gidx, sidx, wvb, acc,
                 si0, si1, sg0, sg1, sg2, sw0, sw1, sw2):
    core = lax.axis_index("c")
    sid = lax.axis_index("s")
    si = [si0, si1]
    sg = [sg0, sg1, sg2]
    sw = [sw0, sw1, sw2]

    if True:
        def fire_i(c, u, _g2=gat2d, _s2=sct2d):
            b2 = sid * IRPT + c * (EC // 128)
            s = si[u % 2]
            pltpu.async_copy(_g2.at[pl.ds(b2, 2)], gidx.at[u], s)
            pltpu.async_copy(_s2.at[pl.ds(b2, 2)], sidx.at[u], s)
            pltpu.async_copy(w2d.at[pl.ds(b2, 2)], wvb.at[u], s)

        def wait_i(u, _g2=gat2d, _s2=sct2d):
            s = si[u % 2]
            pltpu.make_async_copy(_g2.at[pl.ds(0, 2)], gidx.at[u], s).wait()
            pltpu.make_async_copy(_s2.at[pl.ds(0, 2)], sidx.at[u], s).wait()
            pltpu.make_async_copy(w2d.at[pl.ds(0, 2)], wvb.at[u], s).wait()

        def add_off(u, _off):
            for j in range(EC // 128):
                for v in range(8):
                    gidx[u, j, pl.ds(v * 16, 16)] = (
                        gidx[u, j, pl.ds(v * 16, 16)] + _off)

        def fire_g(u, _table=table):
            b3 = u % 3
            for j in range(EC // 128):
                pltpu.async_copy(_table.at[gidx.at[u, j]],
                                 rows.at[b3, pl.ds(j * 128, 128)], sg[b3])

        def wait_g(u, _table=table):
            b3 = u % 3
            for j in range(EC // 128):
                pltpu.make_async_copy(
                    _table.at[gidx.at[u, j]],
                    rows.at[b3, pl.ds(j * 128, 128)], sg[b3]).wait()

        def scale(u):
            b3 = u % 3

            @pl.loop(0, EC // 16)
            def _scale(g16, _u=u, _b3=b3):
                jrow = g16 >> 3
                lo = (g16 & 7) * 16
                w16 = wvb[_u, jrow, pl.ds(lo, 16)]
                base = g16 * 16
                for e in range(16):
                    ws = w16[e]
                    rows[_b3, base + e, pl.ds(0, 16)] = (
                        rows[_b3, base + e, pl.ds(0, 16)] * ws)
                    rows[_b3, base + e, pl.ds(16, 16)] = (
                        rows[_b3, base + e, pl.ds(16, 16)] * ws)

        def fire_w(u):
            b3 = u % 3
            for j in range(EC // 128):
                pltpu.async_copy(rows.at[b3, pl.ds(j * 128, 128)],
                                 acc.at[sidx.at[u, j]], sw[b3], add=True)

        def wait_w(u):
            b3 = u % 3
            for j in range(EC // 128):
                pltpu.make_async_copy(rows.at[b3, pl.ds(j * 128, 128)],
                                      acc.at[sidx.at[u, j]], sw[b3]).wait()

        for k in range(2):
            chunk = core * 2 + k
            off = chunk * trows

            plsc.subcore_barrier()
            pltpu.sync_copy(zeros_h.at[pl.ds(0, nz)],
                            acc.at[pl.ds(sid * nz, nz)])
            plsc.subcore_barrier()

            fire_i(0, 0)
            fire_i(1, 1)
            pltpu.async_copy(zeros_h.at[pl.ds(0, EC)], rows.at[1], sw[1])
            pltpu.async_copy(zeros_h.at[pl.ds(0, EC)], rows.at[2], sw[2])
            wait_i(0)
            add_off(0, off)
            fire_g(0)

            @pl.loop(0, N_EC // 6)
            def _grp(g, _off=off):
                for u in range(6):
                    c = g * 6 + u
                    un1, un2 = (u + 1) % 6, (u + 2) % 6
                    wait_i(un1)
                    add_off(un1, _off)
                    wait_w((u + 4) % 6)
                    fire_g(un1)
                    fire_i(jnp.minimum(c + 2, N_EC - 1), un2)
                    wait_g(u)
                    scale(u)
                    fire_w(u)

            wait_w(4)
            wait_w(5)
            wait_g(0)
            wait_i(1)

            plsc.subcore_barrier()
            pltpu.sync_copy(
                acc.at[pl.ds(sid * nz, nz)],
                out.at[pl.ds(chunk * npad + sid * nz, nz)])


def _sc_spmm_dir(trows, npad, nz, table_flat, gat2d, sct2d, w2d, zeros_h):
    f = pl.kernel(
        functools.partial(_sc_body_dir, trows, npad, nz),
        out_type=jax.ShapeDtypeStruct((NCHUNK * npad, CH), jnp.float32),
        mesh=_MESH,
        compiler_params=pltpu.CompilerParams(use_tc_tiling_on_sc=False),
        scratch_types=[
            pltpu.VMEM((3, EC, CH), jnp.float32),
            pltpu.VMEM((6, EC // 128, 128), jnp.int32),
            pltpu.VMEM((6, EC // 128, 128), jnp.int32),
            pltpu.VMEM((6, EC // 128, 128), jnp.float32),
            pltpu.VMEM_SHARED((npad, CH), jnp.float32),
            pltpu.SemaphoreType.DMA,
            pltpu.SemaphoreType.DMA,
            pltpu.SemaphoreType.DMA,
            pltpu.SemaphoreType.DMA,
            pltpu.SemaphoreType.DMA,
            pltpu.SemaphoreType.DMA,
            pltpu.SemaphoreType.DMA,
            pltpu.SemaphoreType.DMA,
        ],
    )
    return f(table_flat, gat2d, sct2d, w2d, zeros_h)


@jax.jit
def kernel(xs, xg, edge_src_g, edge_dst_s, edge_weight,
           W_s_pre, W_g_pre, W_gs, W_sg, W_s_post, W_g_post):
    e = edge_src_g.shape[0]
    pad = E_PAD - e
    pad_i = jnp.arange(pad, dtype=jnp.int32)
    src2d = jnp.concatenate([edge_src_g, pad_i % NG]).reshape(-1, 128)
    dst2d = jnp.concatenate([edge_dst_s, pad_i % NS]).reshape(-1, 128)
    w2d = jnp.concatenate(
        [edge_weight, jnp.zeros((pad,), jnp.float32)]).reshape(-1, 128)
    zeros_h = jnp.zeros((ZROWS, CH), jnp.float32)

    xs_t = _pre_matmul(xs, W_s_pre)
    xg_t = _pre_matmul(xg, W_g_pre)
    wb_s, wb_g = _fold_weights(W_gs, W_sg, W_s_post, W_g_post)

    agg_s_f = _sc_spmm_dir(NG, NS_PAD, ZROWS,
                           xg_t.reshape(-1, CH), src2d, dst2d, w2d, zeros_h)
    agg_g_f = _sc_spmm_dir(NS, NG_PAD, ZROWS_G,
                           xs_t.reshape(-1, CH), dst2d, src2d, w2d, zeros_h)
    agg_s = agg_s_f.reshape(NCHUNK, NS_PAD, CH)
    agg_g = agg_g_f.reshape(NCHUNK, NG_PAD, CH)

    xs_new = _post_matmul(xs_t, agg_s, W_s_post[:D], wb_s, NS)
    xg_new = _post_matmul(xg_t, agg_g, W_g_post[:D], wb_g, NG)
    return xs_new, xg_new


# speedup vs baseline: 6.8772x; 1.0011x over previous
import functools

import jax
import jax.numpy as jnp
from jax import lax
from jax.experimental import pallas as pl
from jax.experimental.pallas import tpu as pltpu
from jax.experimental.pallas import tpu_sc as plsc

NS, NG, D = 50000, 12500, 128
CH = 32
NCHUNK = D // CH
NS_PAD = 50048
NG_PAD = 12800
N_TILES = 16
EC = 256
N_EC = 126
EPT = N_EC * EC
E_PAD = EPT * N_TILES
IRPT = EPT // 128
ZROWS = NS_PAD // N_TILES
ZROWS_G = NG_PAD // N_TILES
GBYTES = 128 * CH * 4

_MESH = plsc.VectorSubcoreMesh(
    core_axis_name="c", subcore_axis_name="s", num_cores=2, num_subcores=16
)

_GDN = lax.GatherDimensionNumbers(
    offset_dims=(), collapsed_slice_dims=(0,), start_index_map=(0,))


def _pre_body(x_ref, w_ref, out_ref):
    p = jnp.dot(x_ref[...], w_ref[...], preferred_element_type=jnp.float32)
    for c in range(NCHUNK):
        out_ref[c] = p[:, c * CH:(c + 1) * CH]


def _pre_matmul(x, w):
    n = x.shape[0]
    b = 512
    grid = (pl.cdiv(n, b),)
    return pl.pallas_call(
        _pre_body,
        grid=grid,
        in_specs=[
            pl.BlockSpec((b, D), lambda i: (i, 0)),
            pl.BlockSpec((D, D), lambda i: (0, 0)),
        ],
        out_specs=pl.BlockSpec((NCHUNK, b, CH), lambda i: (0, i, 0)),
        out_shape=jax.ShapeDtypeStruct((NCHUNK, n, CH), jnp.float32),
    )(x, w)


def _fold_body(wgs_ref, wsg_ref, wsp_ref, wgp_ref, wbs_ref, wbg_ref):
    wbs_ref[...] = jnp.dot(wgs_ref[...], wsp_ref[D:, :],
                           preferred_element_type=jnp.float32)
    wbg_ref[...] = jnp.dot(wsg_ref[...], wgp_ref[D:, :],
                           preferred_element_type=jnp.float32)


def _fold_weights(w_gs, w_sg, w_s_post, w_g_post):
    return pl.pallas_call(
        _fold_body,
        out_shape=(
            jax.ShapeDtypeStruct((D, D), jnp.float32),
            jax.ShapeDtypeStruct((D, D), jnp.float32),
        ),
    )(w_gs, w_sg, w_s_post, w_g_post)


def _post_body(xp_ref, agg_ref, p1_ref, wb_ref, out_ref):
    acc = jnp.dot(xp_ref[0], p1_ref[0:CH, :], preferred_element_type=jnp.float32)
    for c in range(NCHUNK):
        if c:
            acc += jnp.dot(xp_ref[c], p1_ref[c * CH:(c + 1) * CH, :],
                           preferred_element_type=jnp.float32)
        acc += jnp.dot(agg_ref[c], wb_ref[c * CH:(c + 1) * CH, :],
                       preferred_element_type=jnp.float32)
    out_ref[...] = acc


def _post_matmul(xp, agg, p1, wb, n):
    b = 512
    grid = (pl.cdiv(n, b),)
    return pl.pallas_call(
        _post_body,
        grid=grid,
        in_specs=[
            pl.BlockSpec((NCHUNK, b, CH), lambda i: (0, i, 0)),
            pl.BlockSpec((NCHUNK, b, CH), lambda i: (0, i, 0)),
            pl.BlockSpec((D, D), lambda i: (0, 0)),
            pl.BlockSpec((D, D), lambda i: (0, 0)),
        ],
        out_specs=pl.BlockSpec((b, D), lambda i: (i, 0)),
        out_shape=jax.ShapeDtypeStruct((n, D), jnp.float32),
    )(xp, agg, p1, wb)


def _sc_body_dir(trows, npad, nz,
                 table, gat2d, sct2d, w2d, zeros_h,
                 out,
                 rows, gidx, sidx, wvb, acc,
                 si0, si1, sg0, sg1, sg2, sw0, sw1, sw2):
    core = lax.axis_index("c")
    sid = lax.axis_index("s")
    si = [si0, si1]
    sg = [sg0, sg1, sg2]
    sw = [sw0, sw1, sw2]
    iota16 = lax.iota(jnp.int32, 16)
    bidx = [(iota16 * 0 + e).reshape(16, 1) for e in range(16)]

    if True:
        def fire_i(c, u, _g2=gat2d, _s2=sct2d):
            b2 = sid * IRPT + c * (EC // 128)
            s = si[u % 2]
            pltpu.async_copy(_g2.at[pl.ds(b2, 2)], gidx.at[u], s)
            pltpu.async_copy(_s2.at[pl.ds(b2, 2)], sidx.at[u], s)
            pltpu.async_copy(w2d.at[pl.ds(b2, 2)], wvb.at[u], s)

        def wait_i(u, _g2=gat2d, _s2=sct2d):
            s = si[u % 2]
            pltpu.make_async_copy(_g2.at[pl.ds(0, 2)], gidx.at[u], s).wait()
            pltpu.make_async_copy(_s2.at[pl.ds(0, 2)], sidx.at[u], s).wait()
            pltpu.make_async_copy(w2d.at[pl.ds(0, 2)], wvb.at[u], s).wait()

        def add_off(u, _off):
            for j in range(EC // 128):
                for v in range(8):
                    gidx[u, j, pl.ds(v * 16, 16)] = (
                        gidx[u, j, pl.ds(v * 16, 16)] + _off)

        def fire_g(u, _table=table):
            b3 = u % 3
            for j in range(EC // 128):
                pltpu.async_copy(_table.at[gidx.at[u, j]],
                                 rows.at[b3, pl.ds(j * 128, 128)], sg[b3])

        def wait_g(u, _table=table):
            b3 = u % 3
            for j in range(EC // 128):
                pltpu.make_async_copy(
                    _table.at[gidx.at[u, j]],
                    rows.at[b3, pl.ds(j * 128, 128)], sg[b3]).wait()

        def scale(u):
            b3 = u % 3

            @pl.loop(0, EC // 16)
            def _scale(g16, _u=u, _b3=b3):
                jrow = g16 >> 3
                lo = (g16 & 7) * 16
                w16 = wvb[_u, jrow, pl.ds(lo, 16)]
                base = g16 * 16
                for e in range(16):
                    ws = lax.gather(
                        w16, bidx[e],
                        dimension_numbers=_GDN, slice_sizes=(1,),
                        mode=lax.GatherScatterMode.PROMISE_IN_BOUNDS)
                    rows[_b3, base + e, pl.ds(0, 16)] = (
                        rows[_b3, base + e, pl.ds(0, 16)] * ws)
                    rows[_b3, base + e, pl.ds(16, 16)] = (
                        rows[_b3, base + e, pl.ds(16, 16)] * ws)

        def fire_w(u):
            b3 = u % 3
            for j in range(EC // 128):
                pltpu.async_copy(rows.at[b3, pl.ds(j * 128, 128)],
                                 acc.at[sidx.at[u, j]], sw[b3], add=True)

        def wait_w(u):
            b3 = u % 3
            for j in range(EC // 128):
                pltpu.make_async_copy(rows.at[b3, pl.ds(j * 128, 128)],
                                      acc.at[sidx.at[u, j]], sw[b3]).wait()

        for k in range(2):
            chunk = core * 2 + k
            off = chunk * trows

            plsc.subcore_barrier()
            pltpu.sync_copy(zeros_h.at[pl.ds(0, nz)],
                            acc.at[pl.ds(sid * nz, nz)])
            plsc.subcore_barrier()

            fire_i(0, 0)
            fire_i(1, 1)
            pltpu.async_copy(zeros_h.at[pl.ds(0, EC)], rows.at[1], sw[1])
            pltpu.async_copy(zeros_h.at[pl.ds(0, EC)], rows.at[2], sw[2])
            wait_i(0)
            add_off(0, off)
            fire_g(0)

            @pl.loop(0, N_EC // 6)
            def _grp(g, _off=off):
                for u in range(6):
                    c = g * 6 + u
                    un1, un2 = (u + 1) % 6, (u + 2) % 6
                    wait_i(un1)
                    add_off(un1, _off)
                    wait_w((u + 4) % 6)
                    fire_g(un1)
                    fire_i(jnp.minimum(c + 2, N_EC - 1), un2)
                    wait_g(u)
                    scale(u)
                    fire_w(u)

            wait_w(4)
            wait_w(5)
            wait_g(0)
            wait_i(1)

            plsc.subcore_barrier()
            pltpu.sync_copy(
                acc.at[pl.ds(sid * nz, nz)],
                out.at[pl.ds(chunk * npad + sid * nz, nz)])


def _sc_spmm_dir(trows, npad, nz, table_flat, gat2d, sct2d, w2d, zeros_h):
    f = pl.kernel(
        functools.partial(_sc_body_dir, trows, npad, nz),
        out_type=jax.ShapeDtypeStruct((NCHUNK * npad, CH), jnp.float32),
        mesh=_MESH,
        compiler_params=pltpu.CompilerParams(use_tc_tiling_on_sc=False),
        scratch_types=[
            pltpu.VMEM((3, EC, CH), jnp.float32),
            pltpu.VMEM((6, EC // 128, 128), jnp.int32),
            pltpu.VMEM((6, EC // 128, 128), jnp.int32),
            pltpu.VMEM((6, EC // 128, 128), jnp.float32),
            pltpu.VMEM_SHARED((npad, CH), jnp.float32),
            pltpu.SemaphoreType.DMA,
            pltpu.SemaphoreType.DMA,
            pltpu.SemaphoreType.DMA,
            pltpu.SemaphoreType.DMA,
            pltpu.SemaphoreType.DMA,
            pltpu.SemaphoreType.DMA,
            pltpu.SemaphoreType.DMA,
            pltpu.SemaphoreType.DMA,
        ],
    )
    return f(table_flat, gat2d, sct2d, w2d, zeros_h)


@jax.jit
def kernel(xs, xg, edge_src_g, edge_dst_s, edge_weight,
           W_s_pre, W_g_pre, W_gs, W_sg, W_s_post, W_g_post):
    e = edge_src_g.shape[0]
    pad = E_PAD - e
    pad_i = jnp.arange(pad, dtype=jnp.int32)
    src2d = jnp.concatenate([edge_src_g, pad_i % NG]).reshape(-1, 128)
    dst2d = jnp.concatenate([edge_dst_s, pad_i % NS]).reshape(-1, 128)
    w2d = jnp.concatenate(
        [edge_weight, jnp.zeros((pad,), jnp.float32)]).reshape(-1, 128)
    zeros_h = jnp.zeros((ZROWS, CH), jnp.float32)

    xs_t = _pre_matmul(xs, W_s_pre)
    xg_t = _pre_matmul(xg, W_g_pre)
    wb_s, wb_g = _fold_weights(W_gs, W_sg, W_s_post, W_g_post)

    agg_s_f = _sc_spmm_dir(NG, NS_PAD, ZROWS,
                           xg_t.reshape(-1, CH), src2d, dst2d, w2d, zeros_h)
    agg_g_f = _sc_spmm_dir(NS, NG_PAD, ZROWS_G,
                           xs_t.reshape(-1, CH), dst2d, src2d, w2d, zeros_h)
    agg_s = agg_s_f.reshape(NCHUNK, NS_PAD, CH)
    agg_g = agg_g_f.reshape(NCHUNK, NG_PAD, CH)

    xs_new = _post_matmul(xs_t, agg_s, W_s_post[:D], wb_s, NS)
    xg_new = _post_matmul(xg_t, agg_g, W_g_post[:D], wb_g, NG)
    return xs_new, xg_new


# speedup vs baseline: 6.9830x; 1.0154x over previous
import functools

import jax
import jax.numpy as jnp
from jax import lax
from jax.experimental import pallas as pl
from jax.experimental.pallas import tpu as pltpu
from jax.experimental.pallas import tpu_sc as plsc

NS, NG, D = 50000, 12500, 128
CH = 32
NCHUNK = D // CH
NS_PAD = 50048
NG_PAD = 12800
N_TILES = 16
EC = 256
N_EC = 126
EPT = N_EC * EC
E_PAD = EPT * N_TILES
IRPT = EPT // 128
ZROWS = NS_PAD // N_TILES
ZROWS_G = NG_PAD // N_TILES
GBYTES = 128 * CH * 4

_MESH = plsc.VectorSubcoreMesh(
    core_axis_name="c", subcore_axis_name="s", num_cores=2, num_subcores=16
)

_GDN = lax.GatherDimensionNumbers(
    offset_dims=(), collapsed_slice_dims=(0,), start_index_map=(0,))


def _pre_body(x_ref, w_ref, out_ref):
    p = jnp.dot(x_ref[...], w_ref[...], preferred_element_type=jnp.float32)
    for c in range(NCHUNK):
        out_ref[c] = p[:, c * CH:(c + 1) * CH]


def _pre_matmul(x, w):
    n = x.shape[0]
    b = 512
    grid = (pl.cdiv(n, b),)
    return pl.pallas_call(
        _pre_body,
        grid=grid,
        in_specs=[
            pl.BlockSpec((b, D), lambda i: (i, 0)),
            pl.BlockSpec((D, D), lambda i: (0, 0)),
        ],
        out_specs=pl.BlockSpec((NCHUNK, b, CH), lambda i: (0, i, 0)),
        out_shape=jax.ShapeDtypeStruct((NCHUNK, n, CH), jnp.float32),
    )(x, w)


def _fold_body(wgs_ref, wsg_ref, wsp_ref, wgp_ref, wbs_ref, wbg_ref):
    wbs_ref[...] = jnp.dot(wgs_ref[...], wsp_ref[D:, :],
                           preferred_element_type=jnp.float32)
    wbg_ref[...] = jnp.dot(wsg_ref[...], wgp_ref[D:, :],
                           preferred_element_type=jnp.float32)


def _fold_weights(w_gs, w_sg, w_s_post, w_g_post):
    return pl.pallas_call(
        _fold_body,
        out_shape=(
            jax.ShapeDtypeStruct((D, D), jnp.float32),
            jax.ShapeDtypeStruct((D, D), jnp.float32),
        ),
    )(w_gs, w_sg, w_s_post, w_g_post)


def _post_body(xp_ref, agg_ref, p1_ref, wb_ref, out_ref):
    acc = jnp.dot(xp_ref[0], p1_ref[0:CH, :], preferred_element_type=jnp.float32)
    for c in range(NCHUNK):
        if c:
            acc += jnp.dot(xp_ref[c], p1_ref[c * CH:(c + 1) * CH, :],
                           preferred_element_type=jnp.float32)
        acc += jnp.dot(agg_ref[c], wb_ref[c * CH:(c + 1) * CH, :],
                       preferred_element_type=jnp.float32)
    out_ref[...] = acc


def _post_matmul(xp, agg, p1, wb, n):
    b = 512
    grid = (pl.cdiv(n, b),)
    return pl.pallas_call(
        _post_body,
        grid=grid,
        in_specs=[
            pl.BlockSpec((NCHUNK, b, CH), lambda i: (0, i, 0)),
            pl.BlockSpec((NCHUNK, b, CH), lambda i: (0, i, 0)),
            pl.BlockSpec((D, D), lambda i: (0, 0)),
            pl.BlockSpec((D, D), lambda i: (0, 0)),
        ],
        out_specs=pl.BlockSpec((b, D), lambda i: (i, 0)),
        out_shape=jax.ShapeDtypeStruct((n, D), jnp.float32),
    )(xp, agg, p1, wb)


def _sc_body_dir(trows, npad, nz, ec,
                 table, gat2d, sct2d, w2d, zeros_h,
                 out,
                 rows, gidx, sidx, wvb, acc,
                 si0, si1, sg0, sg1, sg2, sw0, sw1, sw2):
    core = lax.axis_index("c")
    sid = lax.axis_index("s")
    si = [si0, si1]
    sg = [sg0, sg1, sg2]
    sw = [sw0, sw1, sw2]
    iota16 = lax.iota(jnp.int32, 16)
    bidx = [(iota16 * 0 + e).reshape(16, 1) for e in range(16)]

    if True:
        def fire_i(c, u, _g2=gat2d, _s2=sct2d):
            b2 = sid * IRPT + c * (ec // 128)
            s = si[u % 2]
            pltpu.async_copy(_g2.at[pl.ds(b2, ec // 128)], gidx.at[u], s)
            pltpu.async_copy(_s2.at[pl.ds(b2, ec // 128)], sidx.at[u], s)
            pltpu.async_copy(w2d.at[pl.ds(b2, ec // 128)], wvb.at[u], s)

        def wait_i(u, _g2=gat2d, _s2=sct2d):
            s = si[u % 2]
            pltpu.make_async_copy(_g2.at[pl.ds(0, ec // 128)], gidx.at[u], s).wait()
            pltpu.make_async_copy(_s2.at[pl.ds(0, ec // 128)], sidx.at[u], s).wait()
            pltpu.make_async_copy(w2d.at[pl.ds(0, ec // 128)], wvb.at[u], s).wait()

        def add_off(u, _off):
            for j in range(ec // 128):
                for v in range(8):
                    gidx[u, j, pl.ds(v * 16, 16)] = (
                        gidx[u, j, pl.ds(v * 16, 16)] + _off)

        def fire_g(u, _table=table):
            b3 = u % 3
            for j in range(ec // 128):
                pltpu.async_copy(_table.at[gidx.at[u, j]],
                                 rows.at[b3, pl.ds(j * 128, 128)], sg[b3])

        def wait_g(u, _table=table):
            b3 = u % 3
            for j in range(ec // 128):
                pltpu.make_async_copy(
                    _table.at[gidx.at[u, j]],
                    rows.at[b3, pl.ds(j * 128, 128)], sg[b3]).wait()

        def scale(u):
            b3 = u % 3

            @pl.loop(0, ec // 16)
            def _scale(g16, _u=u, _b3=b3):
                jrow = g16 >> 3
                lo = (g16 & 7) * 16
                w16 = wvb[_u, jrow, pl.ds(lo, 16)]
                base = g16 * 16
                for e in range(16):
                    ws = lax.gather(
                        w16, bidx[e],
                        dimension_numbers=_GDN, slice_sizes=(1,),
                        mode=lax.GatherScatterMode.PROMISE_IN_BOUNDS)
                    rows[_b3, base + e, pl.ds(0, 16)] = (
                        rows[_b3, base + e, pl.ds(0, 16)] * ws)
                    rows[_b3, base + e, pl.ds(16, 16)] = (
                        rows[_b3, base + e, pl.ds(16, 16)] * ws)

        def fire_w(u):
            b3 = u % 3
            for j in range(ec // 128):
                pltpu.async_copy(rows.at[b3, pl.ds(j * 128, 128)],
                                 acc.at[sidx.at[u, j]], sw[b3], add=True)

        def wait_w(u):
            b3 = u % 3
            for j in range(ec // 128):
                pltpu.make_async_copy(rows.at[b3, pl.ds(j * 128, 128)],
                                      acc.at[sidx.at[u, j]], sw[b3]).wait()

        for k in range(2):
            chunk = core * 2 + k
            off = chunk * trows

            plsc.subcore_barrier()
            pltpu.sync_copy(zeros_h.at[pl.ds(0, nz)],
                            acc.at[pl.ds(sid * nz, nz)])
            plsc.subcore_barrier()

            fire_i(0, 0)
            fire_i(1, 1)
            pltpu.async_copy(zeros_h.at[pl.ds(0, ec)], rows.at[1], sw[1])
            pltpu.async_copy(zeros_h.at[pl.ds(0, ec)], rows.at[2], sw[2])
            wait_i(0)
            add_off(0, off)
            fire_g(0)

            @pl.loop(0, (EPT // ec) // 6)
            def _grp(g, _off=off):
                for u in range(6):
                    c = g * 6 + u
                    un1, un2 = (u + 1) % 6, (u + 2) % 6
                    wait_i(un1)
                    add_off(un1, _off)
                    wait_w((u + 4) % 6)
                    fire_g(un1)
                    fire_i(jnp.minimum(c + 2, EPT // ec - 1), un2)
                    wait_g(u)
                    scale(u)
                    fire_w(u)

            wait_w(4)
            wait_w(5)
            wait_g(0)
            wait_i(1)

            plsc.subcore_barrier()
            pltpu.sync_copy(
                acc.at[pl.ds(sid * nz, nz)],
                out.at[pl.ds(chunk * npad + sid * nz, nz)])


def _sc_spmm_dir(trows, npad, nz, ec, table_flat, gat2d, sct2d, w2d, zeros_h):
    f = pl.kernel(
        functools.partial(_sc_body_dir, trows, npad, nz, ec),
        out_type=jax.ShapeDtypeStruct((NCHUNK * npad, CH), jnp.float32),
        mesh=_MESH,
        compiler_params=pltpu.CompilerParams(use_tc_tiling_on_sc=False),
        scratch_types=[
            pltpu.VMEM((3, ec, CH), jnp.float32),
            pltpu.VMEM((6, ec // 128, 128), jnp.int32),
            pltpu.VMEM((6, ec // 128, 128), jnp.int32),
            pltpu.VMEM((6, ec // 128, 128), jnp.float32),
            pltpu.VMEM_SHARED((npad, CH), jnp.float32),
            pltpu.SemaphoreType.DMA,
            pltpu.SemaphoreType.DMA,
            pltpu.SemaphoreType.DMA,
            pltpu.SemaphoreType.DMA,
            pltpu.SemaphoreType.DMA,
            pltpu.SemaphoreType.DMA,
            pltpu.SemaphoreType.DMA,
            pltpu.SemaphoreType.DMA,
        ],
    )
    return f(table_flat, gat2d, sct2d, w2d, zeros_h)


@jax.jit
def kernel(xs, xg, edge_src_g, edge_dst_s, edge_weight,
           W_s_pre, W_g_pre, W_gs, W_sg, W_s_post, W_g_post):
    e = edge_src_g.shape[0]
    pad = E_PAD - e
    pad_i = jnp.arange(pad, dtype=jnp.int32)
    src2d = jnp.concatenate([edge_src_g, pad_i % NG]).reshape(-1, 128)
    dst2d = jnp.concatenate([edge_dst_s, pad_i % NS]).reshape(-1, 128)
    w2d = jnp.concatenate(
        [edge_weight, jnp.zeros((pad,), jnp.float32)]).reshape(-1, 128)
    zeros_h = jnp.zeros((ZROWS, CH), jnp.float32)

    xs_t = _pre_matmul(xs, W_s_pre)
    xg_t = _pre_matmul(xg, W_g_pre)
    wb_s, wb_g = _fold_weights(W_gs, W_sg, W_s_post, W_g_post)

    agg_s_f = _sc_spmm_dir(NG, NS_PAD, ZROWS, 256,
                           xg_t.reshape(-1, CH), src2d, dst2d, w2d, zeros_h)
    agg_g_f = _sc_spmm_dir(NS, NG_PAD, ZROWS_G, 768,
                           xs_t.reshape(-1, CH), dst2d, src2d, w2d, zeros_h)
    agg_s = agg_s_f.reshape(NCHUNK, NS_PAD, CH)
    agg_g = agg_g_f.reshape(NCHUNK, NG_PAD, CH)

    xs_new = _post_matmul(xs_t, agg_s, W_s_post[:D], wb_s, NS)
    xg_new = _post_matmul(xg_t, agg_g, W_g_post[:D], wb_g, NG)
    return xs_new, xg_new


# speedup vs baseline: 7.4974x; 1.0737x over previous
import functools

import jax
import jax.numpy as jnp
from jax import lax
from jax.experimental import pallas as pl
from jax.experimental.pallas import tpu as pltpu
from jax.experimental.pallas import tpu_sc as plsc

NS, NG, D = 50000, 12500, 128
CH = 32
NCHUNK = D // CH
NS_PAD = 50048
NG_PAD = 12800
N_TILES = 16
EC = 256
N_EC = 126
EPT = N_EC * EC
E_PAD = EPT * N_TILES
IRPT = EPT // 128
ZROWS = NS_PAD // N_TILES
ZROWS_G = NG_PAD // N_TILES
GBYTES = 128 * CH * 4

_MESH = plsc.VectorSubcoreMesh(
    core_axis_name="c", subcore_axis_name="s", num_cores=2, num_subcores=16
)

_GDN = lax.GatherDimensionNumbers(
    offset_dims=(), collapsed_slice_dims=(0,), start_index_map=(0,))


def _pre_body(x_ref, w_ref, out_ref):
    p = jnp.dot(x_ref[...], w_ref[...], preferred_element_type=jnp.float32)
    for c in range(NCHUNK):
        out_ref[c] = p[:, c * CH:(c + 1) * CH]


def _pre_matmul(x, w):
    n = x.shape[0]
    b = 2048
    grid = (pl.cdiv(n, b),)
    return pl.pallas_call(
        _pre_body,
        grid=grid,
        in_specs=[
            pl.BlockSpec((b, D), lambda i: (i, 0)),
            pl.BlockSpec((D, D), lambda i: (0, 0)),
        ],
        out_specs=pl.BlockSpec((NCHUNK, b, CH), lambda i: (0, i, 0)),
        out_shape=jax.ShapeDtypeStruct((NCHUNK, n, CH), jnp.float32),
    )(x, w)


def _fold_body(wgs_ref, wsg_ref, wsp_ref, wgp_ref, wbs_ref, wbg_ref):
    wbs_ref[...] = jnp.dot(wgs_ref[...], wsp_ref[D:, :],
                           preferred_element_type=jnp.float32)
    wbg_ref[...] = jnp.dot(wsg_ref[...], wgp_ref[D:, :],
                           preferred_element_type=jnp.float32)


def _fold_weights(w_gs, w_sg, w_s_post, w_g_post):
    return pl.pallas_call(
        _fold_body,
        out_shape=(
            jax.ShapeDtypeStruct((D, D), jnp.float32),
            jax.ShapeDtypeStruct((D, D), jnp.float32),
        ),
    )(w_gs, w_sg, w_s_post, w_g_post)


def _post_body(xp_ref, agg_ref, p1_ref, wb_ref, out_ref):
    acc = jnp.dot(xp_ref[0], p1_ref[0:CH, :], preferred_element_type=jnp.float32)
    for c in range(NCHUNK):
        if c:
            acc += jnp.dot(xp_ref[c], p1_ref[c * CH:(c + 1) * CH, :],
                           preferred_element_type=jnp.float32)
        acc += jnp.dot(agg_ref[c], wb_ref[c * CH:(c + 1) * CH, :],
                       preferred_element_type=jnp.float32)
    out_ref[...] = acc


def _post_matmul(xp, agg, p1, wb, n):
    b = 2048
    grid = (pl.cdiv(n, b),)
    return pl.pallas_call(
        _post_body,
        grid=grid,
        in_specs=[
            pl.BlockSpec((NCHUNK, b, CH), lambda i: (0, i, 0)),
            pl.BlockSpec((NCHUNK, b, CH), lambda i: (0, i, 0)),
            pl.BlockSpec((D, D), lambda i: (0, 0)),
            pl.BlockSpec((D, D), lambda i: (0, 0)),
        ],
        out_specs=pl.BlockSpec((b, D), lambda i: (i, 0)),
        out_shape=jax.ShapeDtypeStruct((n, D), jnp.float32),
    )(xp, agg, p1, wb)


def _sc_body_dir(trows, npad, nz, ec,
                 table, gat2d, sct2d, w2d, zeros_h,
                 out,
                 rows, gidx, sidx, wvb, acc,
                 si0, si1, sg0, sg1, sg2, sw0, sw1, sw2):
    core = lax.axis_index("c")
    sid = lax.axis_index("s")
    si = [si0, si1]
    sg = [sg0, sg1, sg2]
    sw = [sw0, sw1, sw2]
    iota16 = lax.iota(jnp.int32, 16)
    bidx = [(iota16 * 0 + e).reshape(16, 1) for e in range(16)]

    if True:
        def fire_i(c, u, _g2=gat2d, _s2=sct2d):
            b2 = sid * IRPT + c * (ec // 128)
            s = si[u % 2]
            pltpu.async_copy(_g2.at[pl.ds(b2, ec // 128)], gidx.at[u], s)
            pltpu.async_copy(_s2.at[pl.ds(b2, ec // 128)], sidx.at[u], s)
            pltpu.async_copy(w2d.at[pl.ds(b2, ec // 128)], wvb.at[u], s)

        def wait_i(u, _g2=gat2d, _s2=sct2d):
            s = si[u % 2]
            pltpu.make_async_copy(_g2.at[pl.ds(0, ec // 128)], gidx.at[u], s).wait()
            pltpu.make_async_copy(_s2.at[pl.ds(0, ec // 128)], sidx.at[u], s).wait()
            pltpu.make_async_copy(w2d.at[pl.ds(0, ec // 128)], wvb.at[u], s).wait()

        def add_off(u, _off):
            for j in range(ec // 128):
                for v in range(8):
                    gidx[u, j, pl.ds(v * 16, 16)] = (
                        gidx[u, j, pl.ds(v * 16, 16)] + _off)

        def fire_g(u, _table=table):
            b3 = u % 3
            for j in range(ec // 128):
                pltpu.async_copy(_table.at[gidx.at[u, j]],
                                 rows.at[b3, pl.ds(j * 128, 128)], sg[b3])

        def wait_g(u, _table=table):
            b3 = u % 3
            for j in range(ec // 128):
                pltpu.make_async_copy(
                    _table.at[gidx.at[u, j]],
                    rows.at[b3, pl.ds(j * 128, 128)], sg[b3]).wait()

        def scale(u):
            b3 = u % 3

            @pl.loop(0, ec // 16)
            def _scale(g16, _u=u, _b3=b3):
                jrow = g16 >> 3
                lo = (g16 & 7) * 16
                w16 = wvb[_u, jrow, pl.ds(lo, 16)]
                base = g16 * 16
                for e in range(16):
                    ws = lax.gather(
                        w16, bidx[e],
                        dimension_numbers=_GDN, slice_sizes=(1,),
                        mode=lax.GatherScatterMode.PROMISE_IN_BOUNDS)
                    rows[_b3, base + e, pl.ds(0, 16)] = (
                        rows[_b3, base + e, pl.ds(0, 16)] * ws)
                    rows[_b3, base + e, pl.ds(16, 16)] = (
                        rows[_b3, base + e, pl.ds(16, 16)] * ws)

        def fire_w(u):
            b3 = u % 3
            for j in range(ec // 128):
                pltpu.async_copy(rows.at[b3, pl.ds(j * 128, 128)],
                                 acc.at[sidx.at[u, j]], sw[b3], add=True)

        def wait_w(u):
            b3 = u % 3
            for j in range(ec // 128):
                pltpu.make_async_copy(rows.at[b3, pl.ds(j * 128, 128)],
                                      acc.at[sidx.at[u, j]], sw[b3]).wait()

        for k in range(2):
            chunk = core * 2 + k
            off = chunk * trows

            plsc.subcore_barrier()
            pltpu.sync_copy(zeros_h.at[pl.ds(0, nz)],
                            acc.at[pl.ds(sid * nz, nz)])
            plsc.subcore_barrier()

            fire_i(0, 0)
            fire_i(1, 1)
            pltpu.async_copy(zeros_h.at[pl.ds(0, ec)], rows.at[1], sw[1])
            pltpu.async_copy(zeros_h.at[pl.ds(0, ec)], rows.at[2], sw[2])
            wait_i(0)
            add_off(0, off)
            fire_g(0)

            @pl.loop(0, (EPT // ec) // 6)
            def _grp(g, _off=off):
                for u in range(6):
                    c = g * 6 + u
                    un1, un2 = (u + 1) % 6, (u + 2) % 6
                    wait_i(un1)
                    add_off(un1, _off)
                    wait_w((u + 4) % 6)
                    fire_g(un1)
                    fire_i(jnp.minimum(c + 2, EPT // ec - 1), un2)
                    wait_g(u)
                    scale(u)
                    fire_w(u)

            wait_w(4)
            wait_w(5)
            wait_g(0)
            wait_i(1)

            plsc.subcore_barrier()
            pltpu.sync_copy(
                acc.at[pl.ds(sid * nz, nz)],
                out.at[pl.ds(chunk * npad + sid * nz, nz)])


def _sc_spmm_dir(trows, npad, nz, ec, table_flat, gat2d, sct2d, w2d, zeros_h):
    f = pl.kernel(
        functools.partial(_sc_body_dir, trows, npad, nz, ec),
        out_type=jax.ShapeDtypeStruct((NCHUNK * npad, CH), jnp.float32),
        mesh=_MESH,
        compiler_params=pltpu.CompilerParams(use_tc_tiling_on_sc=False),
        scratch_types=[
            pltpu.VMEM((3, ec, CH), jnp.float32),
            pltpu.VMEM((6, ec // 128, 128), jnp.int32),
            pltpu.VMEM((6, ec // 128, 128), jnp.int32),
            pltpu.VMEM((6, ec // 128, 128), jnp.float32),
            pltpu.VMEM_SHARED((npad, CH), jnp.float32),
            pltpu.SemaphoreType.DMA,
            pltpu.SemaphoreType.DMA,
            pltpu.SemaphoreType.DMA,
            pltpu.SemaphoreType.DMA,
            pltpu.SemaphoreType.DMA,
            pltpu.SemaphoreType.DMA,
            pltpu.SemaphoreType.DMA,
            pltpu.SemaphoreType.DMA,
        ],
    )
    return f(table_flat, gat2d, sct2d, w2d, zeros_h)


@jax.jit
def kernel(xs, xg, edge_src_g, edge_dst_s, edge_weight,
           W_s_pre, W_g_pre, W_gs, W_sg, W_s_post, W_g_post):
    e = edge_src_g.shape[0]
    pad = E_PAD - e
    pad_i = jnp.arange(pad, dtype=jnp.int32)
    src2d = jnp.concatenate([edge_src_g, pad_i % NG]).reshape(-1, 128)
    dst2d = jnp.concatenate([edge_dst_s, pad_i % NS]).reshape(-1, 128)
    w2d = jnp.concatenate(
        [edge_weight, jnp.zeros((pad,), jnp.float32)]).reshape(-1, 128)
    zeros_h = jnp.zeros((ZROWS, CH), jnp.float32)

    xs_t = _pre_matmul(xs, W_s_pre)
    xg_t = _pre_matmul(xg, W_g_pre)
    wb_s, wb_g = _fold_weights(W_gs, W_sg, W_s_post, W_g_post)

    agg_s_f = _sc_spmm_dir(NG, NS_PAD, ZROWS, 256,
                           xg_t.reshape(-1, CH), src2d, dst2d, w2d, zeros_h)
    agg_g_f = _sc_spmm_dir(NS, NG_PAD, ZROWS_G, 768,
                           xs_t.reshape(-1, CH), dst2d, src2d, w2d, zeros_h)
    agg_s = agg_s_f.reshape(NCHUNK, NS_PAD, CH)
    agg_g = agg_g_f.reshape(NCHUNK, NG_PAD, CH)

    xs_new = _post_matmul(xs_t, agg_s, W_s_post[:D], wb_s, NS)
    xg_new = _post_matmul(xg_t, agg_g, W_g_post[:D], wb_g, NG)
    return xs_new, xg_new
